# baseline (speedup 1.0000x reference)
import functools

import jax
import jax.numpy as jnp
from jax import lax
from jax.experimental import pallas as pl
from jax.experimental.pallas import tpu as pltpu
from jax.experimental.pallas import tpu_sc as plsc

_RADIUS2 = 0.1 * 0.1
_K = 32


_MT = 256
_NT = 512
_NS = 128
_T = 6


def _ballq_kernel(q_ref, c_ref, lt_ref, out_ref, acc_s, cm_s,
                  *, n_points, k, mt, nt):
    b = pl.program_id(0)
    i = pl.program_id(2)
    n_chunks = n_points // nt

    @pl.when(i == 0)
    def _():
        acc_s[...] = jnp.zeros((mt, k), jnp.float32)
        cm_s[...] = jnp.zeros((mt, 1), jnp.float32)

    qx = q_ref[0, :, 0:1]
    qy = q_ref[0, :, 1:2]
    qz = q_ref[0, :, 2:3]
    ik = lax.broadcasted_iota(jnp.int32, (mt, k), 1).astype(jnp.float32)
    nsf = jnp.float32(_NS)

    for s in range(nt // _NS):
        lo, hi = s * _NS, (s + 1) * _NS
        dx = qx - c_ref[0, 0:1, lo:hi]
        dy = qy - c_ref[0, 1:2, lo:hi]
        dz = qz - c_ref[0, 2:3, lo:hi]
        d2 = dx * dx + dy * dy + dz * dz
        msk = jnp.where(d2 <= _RADIUS2, 1.0, 0.0).astype(jnp.float32)
        lc = jnp.dot(msk, lt_ref[...], preferred_element_type=jnp.float32)
        nnz = lc[:, _NS - 1:_NS]
        cmv = cm_s[...]
        tmax = jnp.max(nnz)

        @pl.when(tmax <= jnp.float32(_T))
        def _():
            con = jnp.where(ik >= cmv + nnz, nsf, 0.0)
            for r in range(_T):
                p = jnp.sum(jnp.where(lc <= jnp.float32(r), 1.0, 0.0),
                            axis=1, keepdims=True)
                con = jnp.where(ik == cmv + jnp.float32(r), p, con)
            acc_s[...] += con

        @pl.when(tmax > jnp.float32(_T))
        def _():
            cnt = cmv + lc
            cols = [
                jnp.sum(jnp.where(cnt <= jnp.float32(kk), 1.0, 0.0),
                        axis=1, keepdims=True)
                for kk in range(k)
            ]
            acc_s[...] += jnp.concatenate(cols, axis=1)

        cm_s[...] = cmv + nnz

    @pl.when(i == n_chunks - 1)
    def _():
        acc = acc_s[...]
        nf = jnp.float32(n_points)
        first = acc[:, 0:1]
        first = jnp.where(first >= nf, 0.0, first)
        idx = jnp.where(acc >= nf, first, acc)
        out_ref[0, :, :] = idx.astype(jnp.int32) + b * n_points


def _ball_query(queries, coords_t):
    B, M, _ = queries.shape
    N = coords_t.shape[2]
    ra = lax.broadcasted_iota(jnp.int32, (_NS, _NS), 0)
    rb = lax.broadcasted_iota(jnp.int32, (_NS, _NS), 1)
    lt = jnp.where(ra <= rb, 1.0, 0.0).astype(jnp.float32)
    grid = (B, M // _MT, N // _NT)
    return pl.pallas_call(
        functools.partial(_ballq_kernel, n_points=N, k=_K, mt=_MT, nt=_NT),
        grid=grid,
        in_specs=[
            pl.BlockSpec((1, _MT, 3), lambda b, m, i: (b, m, 0)),
            pl.BlockSpec((1, 3, _NT), lambda b, m, i: (b, 0, i)),
            pl.BlockSpec((_NS, _NS), lambda b, m, i: (0, 0)),
        ],
        out_specs=pl.BlockSpec((1, _MT, _K), lambda b, m, i: (b, m, 0)),
        out_shape=jax.ShapeDtypeStruct((B, M, _K), jnp.int32),
        scratch_shapes=[
            pltpu.VMEM((_MT, _K), jnp.float32),
            pltpu.VMEM((_MT, 1), jnp.float32),
        ],
    )(queries, coords_t, lt)


_D = 128
_QW = 16
_RC = 128


def _sc_gather(table, gidx, qrep):
    rows = gidx.shape[0]
    info = plsc.get_sparse_core_info()
    nc, ns = info.num_cores, info.num_subcores
    nw = nc * ns
    per_w = rows // nw
    n_chunks = per_w // _RC
    mesh = plsc.VectorSubcoreMesh(core_axis_name="c", subcore_axis_name="s")

    @functools.partial(
        pl.kernel,
        mesh=mesh,
        out_type=jax.ShapeDtypeStruct((rows, _D), jnp.float32),
        scratch_types=[
            pltpu.VMEM((_RC,), jnp.int32),
            pltpu.VMEM((_RC, _D), jnp.float32),
            pltpu.VMEM((_RC, _QW), jnp.float32),
            pltpu.SemaphoreType.DMA,
        ],
    )
    def k(table_hbm, gidx_hbm, qrep_hbm, out_hbm, idx_v, rows_v, q_v, sem):
        wid = lax.axis_index("s") * nc + lax.axis_index("c")
        base = wid * per_w

        def chunk(t, _):
            gbase = base + t * _RC
            pltpu.sync_copy(gidx_hbm.at[pl.ds(gbase, _RC)], idx_v)
            pltpu.async_copy(table_hbm.at[idx_v], rows_v, sem).wait()
            pltpu.sync_copy(qrep_hbm.at[pl.ds(gbase, _RC)], q_v)

            def sub(r, _):
                rows_v[r, 0:_QW] = rows_v[r, 0:_QW] - q_v[r, :]
                return ()

            lax.fori_loop(0, _RC, sub, ())
            pltpu.sync_copy(rows_v, out_hbm.at[pl.ds(gbase, _RC)])
            return ()

        lax.fori_loop(0, n_chunks, chunk, ())

    return k(table, gidx, qrep)


def kernel(coords, features, t_embed, queries):
    B, N, _ = coords.shape
    M = queries.shape[1]
    C = features.shape[1]
    Ct = t_embed.shape[1]

    coords_t = jnp.transpose(coords, (0, 2, 1))
    gidx = _ball_query(queries, coords_t).reshape(B * M * _K)

    pad = _D - 3 - C - Ct
    table = jnp.concatenate(
        [coords,
         jnp.transpose(features, (0, 2, 1)),
         jnp.transpose(t_embed, (0, 2, 1)),
         jnp.zeros((B, N, pad), jnp.float32)],
        axis=-1).reshape(B * N, _D)

    qpad = jnp.concatenate(
        [queries, jnp.zeros((B, M, _QW - 3), jnp.float32)], axis=-1)
    qrep = jnp.broadcast_to(qpad[:, :, None, :],
                            (B, M, _K, _QW)).reshape(B * M * _K, _QW)

    g = _sc_gather(table, gidx, qrep).reshape(B, M, _K, _D)
    grouped_features = jnp.transpose(g[..., 0:3 + C], (0, 3, 1, 2))
    gt = jnp.transpose(g[..., 3 + C:3 + C + Ct], (0, 3, 1, 2))
    return (grouped_features, gt)


# speedup vs baseline: 1.4193x; 1.4193x over previous
import functools

import jax
import jax.numpy as jnp
from jax import lax
from jax.experimental import pallas as pl
from jax.experimental.pallas import tpu as pltpu
from jax.experimental.pallas import tpu_sc as plsc

_RADIUS2 = 0.1 * 0.1
_K = 32


_MT = 128
_NPC = 1024
_NS = 256
_T = 6


def _ballq_kernel(q_ref, c_ref, lt_ref, out_ref, acc_s, cm_s,
                  *, n_points, k, mt, npc):
    b = pl.program_id(0)
    i = pl.program_id(2)
    n_chunks = n_points // npc

    @pl.when(i == 0)
    def _():
        acc_s[...] = jnp.zeros((k, mt), jnp.float32)
        cm_s[...] = jnp.zeros((1, mt), jnp.float32)

    qx = q_ref[0, 0:1, :]
    qy = q_ref[0, 1:2, :]
    qz = q_ref[0, 2:3, :]
    ik = lax.broadcasted_iota(jnp.int32, (k, mt), 0).astype(jnp.float32)
    nsf = jnp.float32(_NS)

    for s in range(npc // _NS):
        lo, hi = s * _NS, (s + 1) * _NS
        dx = c_ref[0, lo:hi, 0:1] - qx
        dy = c_ref[0, lo:hi, 1:2] - qy
        dz = c_ref[0, lo:hi, 2:3] - qz
        d2 = dx * dx + dy * dy + dz * dz
        msk = jnp.where(d2 <= _RADIUS2, 1.0, 0.0).astype(jnp.float32)
        lc = jnp.dot(lt_ref[...], msk, preferred_element_type=jnp.float32)
        nnz = lc[_NS - 1:_NS, :]
        cmv = cm_s[...]
        tmax = jnp.max(nnz)

        @pl.when(tmax <= jnp.float32(_T))
        def _():
            con = jnp.where(ik >= cmv + nnz, nsf, 0.0)
            for r in range(_T):
                p = jnp.sum(jnp.where(lc <= jnp.float32(r), 1.0, 0.0),
                            axis=0, keepdims=True)
                con = jnp.where(ik == cmv + jnp.float32(r), p, con)
            acc_s[...] += con

        @pl.when(tmax > jnp.float32(_T))
        def _():
            cnt = cmv + lc
            rows = [
                jnp.sum(jnp.where(cnt <= jnp.float32(kk), 1.0, 0.0),
                        axis=0, keepdims=True)
                for kk in range(k)
            ]
            acc_s[...] += jnp.concatenate(rows, axis=0)

        cm_s[...] = cmv + nnz

    @pl.when(i == n_chunks - 1)
    def _():
        acc = acc_s[...]
        nf = jnp.float32(n_points)
        first = acc[0:1, :]
        first = jnp.where(first >= nf, 0.0, first)
        idx = jnp.where(acc >= nf, first, acc)
        out_ref[0, :, :] = idx.astype(jnp.int32) + b * n_points


def _ball_query(queries_t, coords):
    B, N, _ = coords.shape
    M = queries_t.shape[2]
    ra = lax.broadcasted_iota(jnp.int32, (_NS, _NS), 0)
    rb = lax.broadcasted_iota(jnp.int32, (_NS, _NS), 1)
    lt = jnp.where(rb <= ra, 1.0, 0.0).astype(jnp.float32)
    grid = (B, M // _MT, N // _NPC)
    return pl.pallas_call(
        functools.partial(_ballq_kernel, n_points=N, k=_K, mt=_MT, npc=_NPC),
        grid=grid,
        in_specs=[
            pl.BlockSpec((1, 3, _MT), lambda b, m, i: (b, 0, m)),
            pl.BlockSpec((1, _NPC, 3), lambda b, m, i: (b, i, 0)),
            pl.BlockSpec((_NS, _NS), lambda b, m, i: (0, 0)),
        ],
        out_specs=pl.BlockSpec((1, _K, _MT), lambda b, m, i: (b, 0, m)),
        out_shape=jax.ShapeDtypeStruct((B, _K, M), jnp.int32),
        scratch_shapes=[
            pltpu.VMEM((_K, _MT), jnp.float32),
            pltpu.VMEM((1, _MT), jnp.float32),
        ],
    )(queries_t, coords, lt)


_D = 128
_QW = 16
_RC = 128


def _sc_gather(table, gidx, qrep):
    rows = gidx.shape[0]
    info = plsc.get_sparse_core_info()
    nc, ns = info.num_cores, info.num_subcores
    nw = nc * ns
    per_w = rows // nw
    n_chunks = per_w // _RC
    mesh = plsc.VectorSubcoreMesh(core_axis_name="c", subcore_axis_name="s")

    @functools.partial(
        pl.kernel,
        mesh=mesh,
        out_type=jax.ShapeDtypeStruct((rows, _D), jnp.float32),
        scratch_types=[
            pltpu.VMEM((_RC,), jnp.int32),
            pltpu.VMEM((_RC, _D), jnp.float32),
            pltpu.VMEM((_RC, _QW), jnp.float32),
            pltpu.SemaphoreType.DMA,
        ],
    )
    def k(table_hbm, gidx_hbm, qrep_hbm, out_hbm, idx_v, rows_v, q_v, sem):
        wid = lax.axis_index("s") * nc + lax.axis_index("c")
        base = wid * per_w

        def chunk(t, _):
            gbase = base + t * _RC
            pltpu.sync_copy(gidx_hbm.at[pl.ds(gbase, _RC)], idx_v)
            pltpu.async_copy(table_hbm.at[idx_v], rows_v, sem).wait()
            pltpu.sync_copy(qrep_hbm.at[pl.ds(gbase, _RC)], q_v)

            def sub(r, _):
                rows_v[r, 0:_QW] = rows_v[r, 0:_QW] - q_v[r, :]
                return ()

            lax.fori_loop(0, _RC, sub, ())
            pltpu.sync_copy(rows_v, out_hbm.at[pl.ds(gbase, _RC)])
            return ()

        lax.fori_loop(0, n_chunks, chunk, ())

    return k(table, gidx, qrep)


def kernel(coords, features, t_embed, queries):
    B, N, _ = coords.shape
    M = queries.shape[1]
    C = features.shape[1]
    Ct = t_embed.shape[1]

    queries_t = jnp.transpose(queries, (0, 2, 1))
    gidx = jnp.transpose(_ball_query(queries_t, coords),
                         (0, 2, 1)).reshape(B * M * _K)

    pad = _D - 3 - C - Ct
    table = jnp.concatenate(
        [coords,
         jnp.transpose(features, (0, 2, 1)),
         jnp.transpose(t_embed, (0, 2, 1)),
         jnp.zeros((B, N, pad), jnp.float32)],
        axis=-1).reshape(B * N, _D)

    qpad = jnp.concatenate(
        [queries, jnp.zeros((B, M, _QW - 3), jnp.float32)], axis=-1)
    qrep = jnp.broadcast_to(qpad[:, :, None, :],
                            (B, M, _K, _QW)).reshape(B * M * _K, _QW)

    g = _sc_gather(table, gidx, qrep).reshape(B, M, _K, _D)
    grouped_features = jnp.transpose(g[..., 0:3 + C], (0, 3, 1, 2))
    gt = jnp.transpose(g[..., 3 + C:3 + C + Ct], (0, 3, 1, 2))
    return (grouped_features, gt)


# speedup vs baseline: 1.5680x; 1.1048x over previous
import functools

import jax
import jax.numpy as jnp
from jax import lax
from jax.experimental import pallas as pl
from jax.experimental.pallas import tpu as pltpu
from jax.experimental.pallas import tpu_sc as plsc

_RADIUS2 = 0.1 * 0.1
_K = 32


_MT = 128
_NPC = 1024
_NS = 256
_T = 6


def _ballq_kernel(q_ref, c_ref, lt_ref, out_ref, acc_s, cm_s,
                  *, n_points, k, mt, npc):
    b = pl.program_id(0)
    i = pl.program_id(2)
    n_chunks = n_points // npc

    @pl.when(i == 0)
    def _():
        acc_s[...] = jnp.zeros((k, mt), jnp.float32)
        cm_s[...] = jnp.zeros((1, mt), jnp.float32)

    qx = q_ref[0, 0:1, :]
    qy = q_ref[0, 1:2, :]
    qz = q_ref[0, 2:3, :]
    ik = lax.broadcasted_iota(jnp.int32, (k, mt), 0).astype(jnp.float32)
    nsf = jnp.float32(_NS)

    for s in range(npc // _NS):
        lo, hi = s * _NS, (s + 1) * _NS
        dx = c_ref[0, lo:hi, 0:1] - qx
        dy = c_ref[0, lo:hi, 1:2] - qy
        dz = c_ref[0, lo:hi, 2:3] - qz
        d2 = dx * dx + dy * dy + dz * dz
        msk = jnp.where(d2 <= _RADIUS2, 1.0, 0.0).astype(jnp.float32)
        lc = jnp.dot(lt_ref[...], msk, preferred_element_type=jnp.float32)
        nnz = lc[_NS - 1:_NS, :]
        cmv = cm_s[...]
        tmax = jnp.max(nnz)

        con = jnp.where(ik >= cmv + nnz, nsf, 0.0)
        for r in range(_T):
            p = jnp.sum(jnp.where(lc <= jnp.float32(r), 1.0, 0.0),
                        axis=0, keepdims=True)
            con = jnp.where(ik == cmv + jnp.float32(r), p, con)
        acc_s[...] += con

        def more(w):
            return w.astype(jnp.float32) * jnp.float32(_T) < tmax

        def wave(w):
            basef = w.astype(jnp.float32) * jnp.float32(_T)
            delta = jnp.zeros((k, mt), jnp.float32)
            for dr in range(_T):
                r = basef + jnp.float32(dr)
                p = jnp.sum(jnp.where(lc <= r, 1.0, 0.0),
                            axis=0, keepdims=True)
                delta = jnp.where((ik == cmv + r) & (nnz > r), p, delta)
            acc_s[...] += delta
            return w + 1

        lax.while_loop(more, wave, jnp.int32(1))
        cm_s[...] = cmv + nnz

    @pl.when(i == n_chunks - 1)
    def _():
        acc = acc_s[...]
        nf = jnp.float32(n_points)
        first = acc[0:1, :]
        first = jnp.where(first >= nf, 0.0, first)
        idx = jnp.where(acc >= nf, first, acc)
        out_ref[0, :, :] = idx.astype(jnp.int32) + b * n_points


def _ball_query(queries_t, coords):
    B, N, _ = coords.shape
    M = queries_t.shape[2]
    ra = lax.broadcasted_iota(jnp.int32, (_NS, _NS), 0)
    rb = lax.broadcasted_iota(jnp.int32, (_NS, _NS), 1)
    lt = jnp.where(rb <= ra, 1.0, 0.0).astype(jnp.float32)
    grid = (B, M // _MT, N // _NPC)
    return pl.pallas_call(
        functools.partial(_ballq_kernel, n_points=N, k=_K, mt=_MT, npc=_NPC),
        grid=grid,
        in_specs=[
            pl.BlockSpec((1, 3, _MT), lambda b, m, i: (b, 0, m)),
            pl.BlockSpec((1, _NPC, 3), lambda b, m, i: (b, i, 0)),
            pl.BlockSpec((_NS, _NS), lambda b, m, i: (0, 0)),
        ],
        out_specs=pl.BlockSpec((1, _K, _MT), lambda b, m, i: (b, 0, m)),
        out_shape=jax.ShapeDtypeStruct((B, _K, M), jnp.int32),
        scratch_shapes=[
            pltpu.VMEM((_K, _MT), jnp.float32),
            pltpu.VMEM((1, _MT), jnp.float32),
        ],
    )(queries_t, coords, lt)


_D = 128
_QW = 16
_RC = 128


def _sc_gather(table, gidx, qrep):
    rows = gidx.shape[0]
    info = plsc.get_sparse_core_info()
    nc, ns = info.num_cores, info.num_subcores
    nw = nc * ns
    per_w = rows // nw
    n_chunks = per_w // _RC
    mesh = plsc.VectorSubcoreMesh(core_axis_name="c", subcore_axis_name="s")

    @functools.partial(
        pl.kernel,
        mesh=mesh,
        out_type=jax.ShapeDtypeStruct((rows, _D), jnp.float32),
        scratch_types=[
            pltpu.VMEM((_RC,), jnp.int32),
            pltpu.VMEM((_RC, _D), jnp.float32),
            pltpu.VMEM((_RC, _QW), jnp.float32),
            pltpu.SemaphoreType.DMA,
        ],
    )
    def k(table_hbm, gidx_hbm, qrep_hbm, out_hbm, idx_v, rows_v, q_v, sem):
        wid = lax.axis_index("s") * nc + lax.axis_index("c")
        base = wid * per_w

        def chunk(t, _):
            gbase = base + t * _RC
            pltpu.sync_copy(gidx_hbm.at[pl.ds(gbase, _RC)], idx_v)
            pltpu.async_copy(table_hbm.at[idx_v], rows_v, sem).wait()
            pltpu.sync_copy(qrep_hbm.at[pl.ds(gbase, _RC)], q_v)

            def sub(r, _):
                rows_v[r, 0:_QW] = rows_v[r, 0:_QW] - q_v[r, :]
                return ()

            lax.fori_loop(0, _RC, sub, ())
            pltpu.sync_copy(rows_v, out_hbm.at[pl.ds(gbase, _RC)])
            return ()

        lax.fori_loop(0, n_chunks, chunk, ())

    return k(table, gidx, qrep)


def kernel(coords, features, t_embed, queries):
    B, N, _ = coords.shape
    M = queries.shape[1]
    C = features.shape[1]
    Ct = t_embed.shape[1]

    queries_t = jnp.transpose(queries, (0, 2, 1))
    gidx = jnp.transpose(_ball_query(queries_t, coords),
                         (0, 2, 1)).reshape(B * M * _K)

    pad = _D - 3 - C - Ct
    table = jnp.concatenate(
        [coords,
         jnp.transpose(features, (0, 2, 1)),
         jnp.transpose(t_embed, (0, 2, 1)),
         jnp.zeros((B, N, pad), jnp.float32)],
        axis=-1).reshape(B * N, _D)

    qpad = jnp.concatenate(
        [queries, jnp.zeros((B, M, _QW - 3), jnp.float32)], axis=-1)
    qrep = jnp.broadcast_to(qpad[:, :, None, :],
                            (B, M, _K, _QW)).reshape(B * M * _K, _QW)

    g = _sc_gather(table, gidx, qrep).reshape(B, M, _K, _D)
    grouped_features = jnp.transpose(g[..., 0:3 + C], (0, 3, 1, 2))
    gt = jnp.transpose(g[..., 3 + C:3 + C + Ct], (0, 3, 1, 2))
    return (grouped_features, gt)


# speedup vs baseline: 2.5595x; 1.6324x over previous
import functools

import jax
import jax.numpy as jnp
from jax import lax
from jax.experimental import pallas as pl
from jax.experimental.pallas import tpu as pltpu
from jax.experimental.pallas import tpu_sc as plsc

_RADIUS2 = 0.1 * 0.1
_K = 32


_MT = 128
_NPC = 1024
_NS = 256
_T = 6


def _ballq_kernel(q_ref, c_ref, lt_ref, out_ref, acc_s, cm_s,
                  *, n_points, k, mt, npc):
    b = pl.program_id(0)
    i = pl.program_id(2)
    n_chunks = n_points // npc

    @pl.when(i == 0)
    def _():
        acc_s[...] = jnp.zeros((k, mt), jnp.float32)
        cm_s[...] = jnp.zeros((1, mt), jnp.float32)

    qx = q_ref[0, 0:1, :]
    qy = q_ref[0, 1:2, :]
    qz = q_ref[0, 2:3, :]
    ik = lax.broadcasted_iota(jnp.int32, (k, mt), 0).astype(jnp.float32)
    nsf = jnp.float32(_NS)

    def sub_lc(s):
        lo, hi = s * _NS, (s + 1) * _NS
        dx = c_ref[0, lo:hi, 0:1] - qx
        dy = c_ref[0, lo:hi, 1:2] - qy
        dz = c_ref[0, lo:hi, 2:3] - qz
        d2 = dx * dx + dy * dy + dz * dz
        msk = jnp.where(d2 <= _RADIUS2, 1.0, 0.0).astype(jnp.float32)
        return jnp.dot(lt_ref[...], msk, preferred_element_type=jnp.float32)

    cm0 = cm_s[...]
    nmax = jnp.zeros((1, mt), jnp.float32)
    cmv = cm0
    for s in range(npc // _NS):
        lc = sub_lc(s)
        nnz = lc[_NS - 1:_NS, :]
        con = jnp.where(ik >= cmv + nnz, nsf, 0.0)
        for r in range(_T):
            p = jnp.sum(jnp.where(lc <= jnp.float32(r), 1.0, 0.0),
                        axis=0, keepdims=True)
            con = jnp.where(ik == cmv + jnp.float32(r), p, con)
        acc_s[...] += con
        nmax = jnp.maximum(nmax, nnz)
        cmv = cmv + nnz
    cm_s[...] = cmv

    tmax = jnp.max(nmax)

    def more(w):
        return w.astype(jnp.float32) * jnp.float32(_T) < tmax

    def wave(w):
        basef = w.astype(jnp.float32) * jnp.float32(_T)
        cmw = cm0
        for s in range(npc // _NS):
            lc = sub_lc(s)
            nnz = lc[_NS - 1:_NS, :]
            delta = jnp.zeros((k, mt), jnp.float32)
            for dr in range(_T):
                r = basef + jnp.float32(dr)
                p = jnp.sum(jnp.where(lc <= r, 1.0, 0.0),
                            axis=0, keepdims=True)
                delta = jnp.where((ik == cmw + r) & (nnz > r), p, delta)
            acc_s[...] += delta
            cmw = cmw + nnz
        return w + 1

    lax.while_loop(more, wave, jnp.int32(1))

    @pl.when(i == n_chunks - 1)
    def _():
        acc = acc_s[...]
        nf = jnp.float32(n_points)
        first = acc[0:1, :]
        first = jnp.where(first >= nf, 0.0, first)
        idx = jnp.where(acc >= nf, first, acc)
        out_ref[0, :, :] = idx.astype(jnp.int32) + b * n_points


def _ball_query(queries_t, coords):
    B, N, _ = coords.shape
    M = queries_t.shape[2]
    ra = lax.broadcasted_iota(jnp.int32, (_NS, _NS), 0)
    rb = lax.broadcasted_iota(jnp.int32, (_NS, _NS), 1)
    lt = jnp.where(rb <= ra, 1.0, 0.0).astype(jnp.float32)
    grid = (B, M // _MT, N // _NPC)
    return pl.pallas_call(
        functools.partial(_ballq_kernel, n_points=N, k=_K, mt=_MT, npc=_NPC),
        grid=grid,
        in_specs=[
            pl.BlockSpec((1, 3, _MT), lambda b, m, i: (b, 0, m)),
            pl.BlockSpec((1, _NPC, 3), lambda b, m, i: (b, i, 0)),
            pl.BlockSpec((_NS, _NS), lambda b, m, i: (0, 0)),
        ],
        out_specs=pl.BlockSpec((1, _K, _MT), lambda b, m, i: (b, 0, m)),
        out_shape=jax.ShapeDtypeStruct((B, _K, M), jnp.int32),
        scratch_shapes=[
            pltpu.VMEM((_K, _MT), jnp.float32),
            pltpu.VMEM((1, _MT), jnp.float32),
        ],
    )(queries_t, coords, lt)


_D = 128
_QW = 16
_RC = 128


def _sc_gather(table, gidx, qrep):
    rows = gidx.shape[0]
    info = plsc.get_sparse_core_info()
    nc, ns = info.num_cores, info.num_subcores
    nw = nc * ns
    per_w = rows // nw
    n_chunks = per_w // _RC
    mesh = plsc.VectorSubcoreMesh(core_axis_name="c", subcore_axis_name="s")

    @functools.partial(
        pl.kernel,
        mesh=mesh,
        out_type=jax.ShapeDtypeStruct((rows, _D), jnp.float32),
        scratch_types=[
            pltpu.VMEM((_RC,), jnp.int32),
            pltpu.VMEM((_RC, _D), jnp.float32),
            pltpu.VMEM((_RC, _QW), jnp.float32),
            pltpu.SemaphoreType.DMA,
        ],
    )
    def k(table_hbm, gidx_hbm, qrep_hbm, out_hbm, idx_v, rows_v, q_v, sem):
        wid = lax.axis_index("s") * nc + lax.axis_index("c")
        base = wid * per_w

        def chunk(t, _):
            gbase = base + t * _RC
            pltpu.sync_copy(gidx_hbm.at[pl.ds(gbase, _RC)], idx_v)
            pltpu.async_copy(table_hbm.at[idx_v], rows_v, sem).wait()
            pltpu.sync_copy(qrep_hbm.at[pl.ds(gbase, _RC)], q_v)

            def sub(r, _):
                rows_v[r, 0:_QW] = rows_v[r, 0:_QW] - q_v[r, :]
                return ()

            lax.fori_loop(0, _RC, sub, ())
            pltpu.sync_copy(rows_v, out_hbm.at[pl.ds(gbase, _RC)])
            return ()

        lax.fori_loop(0, n_chunks, chunk, ())

    return k(table, gidx, qrep)


def kernel(coords, features, t_embed, queries):
    B, N, _ = coords.shape
    M = queries.shape[1]
    C = features.shape[1]
    Ct = t_embed.shape[1]

    queries_t = jnp.transpose(queries, (0, 2, 1))
    gidx = jnp.transpose(_ball_query(queries_t, coords),
                         (0, 2, 1)).reshape(B * M * _K)

    pad = _D - 3 - C - Ct
    table = jnp.concatenate(
        [coords,
         jnp.transpose(features, (0, 2, 1)),
         jnp.transpose(t_embed, (0, 2, 1)),
         jnp.zeros((B, N, pad), jnp.float32)],
        axis=-1).reshape(B * N, _D)

    qpad = jnp.concatenate(
        [queries, jnp.zeros((B, M, _QW - 3), jnp.float32)], axis=-1)
    qrep = jnp.broadcast_to(qpad[:, :, None, :],
                            (B, M, _K, _QW)).reshape(B * M * _K, _QW)

    g = _sc_gather(table, gidx, qrep).reshape(B, M, _K, _D)
    grouped_features = jnp.transpose(g[..., 0:3 + C], (0, 3, 1, 2))
    gt = jnp.transpose(g[..., 3 + C:3 + C + Ct], (0, 3, 1, 2))
    return (grouped_features, gt)


# speedup vs baseline: 2.6447x; 1.0333x over previous
import functools

import jax
import jax.numpy as jnp
from jax import lax
from jax.experimental import pallas as pl
from jax.experimental.pallas import tpu as pltpu
from jax.experimental.pallas import tpu_sc as plsc

_RADIUS2 = 0.1 * 0.1
_K = 32


_MT = 128
_NPC = 1024
_NS = 256
_T = 6


def _ballq_kernel(q_ref, c_ref, lt_ref, out_ref, acc_s, cm_s,
                  *, n_points, k, mt, npc):
    b = pl.program_id(0)
    i = pl.program_id(2)
    n_chunks = n_points // npc

    @pl.when(i == 0)
    def _():
        acc_s[...] = jnp.zeros((k, mt), jnp.float32)
        cm_s[...] = jnp.zeros((1, mt), jnp.float32)

    qx = q_ref[0, 0:1, :]
    qy = q_ref[0, 1:2, :]
    qz = q_ref[0, 2:3, :]
    ik = lax.broadcasted_iota(jnp.int32, (k, mt), 0).astype(jnp.float32)
    nsf = jnp.float32(_NS)

    def sub_lc(s):
        lo, hi = s * _NS, (s + 1) * _NS
        dx = c_ref[0, lo:hi, 0:1] - qx
        dy = c_ref[0, lo:hi, 1:2] - qy
        dz = c_ref[0, lo:hi, 2:3] - qz
        d2 = dx * dx + dy * dy + dz * dz
        msk = jnp.where(d2 <= _RADIUS2, 1.0, 0.0).astype(jnp.float32)
        return jnp.dot(lt_ref[...], msk, preferred_element_type=jnp.float32)

    cm0 = cm_s[...]
    nmax = jnp.zeros((1, mt), jnp.float32)
    cmv = cm0
    for s in range(npc // _NS):
        lc = sub_lc(s)
        nnz = lc[_NS - 1:_NS, :]
        con = jnp.where(ik >= cmv + nnz, nsf, 0.0)
        for r in range(_T):
            p = jnp.sum(jnp.where(lc <= jnp.float32(r), 1.0, 0.0),
                        axis=0, keepdims=True)
            con = jnp.where(ik == cmv + jnp.float32(r), p, con)
        acc_s[...] += con
        nmax = jnp.maximum(nmax, nnz)
        cmv = cmv + nnz
    cm_s[...] = cmv

    tmax = jnp.max(nmax)

    def more(w):
        return w.astype(jnp.float32) * jnp.float32(_T) < tmax

    def wave(w):
        basef = w.astype(jnp.float32) * jnp.float32(_T)
        cmw = cm0
        for s in range(npc // _NS):
            lc = sub_lc(s)
            nnz = lc[_NS - 1:_NS, :]
            delta = jnp.zeros((k, mt), jnp.float32)
            for dr in range(_T):
                r = basef + jnp.float32(dr)
                p = jnp.sum(jnp.where(lc <= r, 1.0, 0.0),
                            axis=0, keepdims=True)
                delta = jnp.where((ik == cmw + r) & (nnz > r), p, delta)
            acc_s[...] += delta
            cmw = cmw + nnz
        return w + 1

    lax.while_loop(more, wave, jnp.int32(1))

    @pl.when(i == n_chunks - 1)
    def _():
        acc = acc_s[...]
        nf = jnp.float32(n_points)
        first = acc[0:1, :]
        first = jnp.where(first >= nf, 0.0, first)
        idx = jnp.where(acc >= nf, first, acc)
        out_ref[0, :, :] = idx.astype(jnp.int32) + b * n_points


def _ball_query(queries_t, coords):
    B, N, _ = coords.shape
    M = queries_t.shape[2]
    ra = lax.broadcasted_iota(jnp.int32, (_NS, _NS), 0)
    rb = lax.broadcasted_iota(jnp.int32, (_NS, _NS), 1)
    lt = jnp.where(rb <= ra, 1.0, 0.0).astype(jnp.float32)
    grid = (B, M // _MT, N // _NPC)
    return pl.pallas_call(
        functools.partial(_ballq_kernel, n_points=N, k=_K, mt=_MT, npc=_NPC),
        grid=grid,
        in_specs=[
            pl.BlockSpec((1, 3, _MT), lambda b, m, i: (b, 0, m)),
            pl.BlockSpec((1, _NPC, 3), lambda b, m, i: (b, i, 0)),
            pl.BlockSpec((_NS, _NS), lambda b, m, i: (0, 0)),
        ],
        out_specs=pl.BlockSpec((1, _K, _MT), lambda b, m, i: (b, 0, m)),
        out_shape=jax.ShapeDtypeStruct((B, _K, M), jnp.int32),
        scratch_shapes=[
            pltpu.VMEM((_K, _MT), jnp.float32),
            pltpu.VMEM((1, _MT), jnp.float32),
        ],
    )(queries_t, coords, lt)


_D = 128
_QW = 16
_RC = 128


def _sc_gather(table, gidx, qrep):
    rows = gidx.shape[0]
    info = plsc.get_sparse_core_info()
    nc, ns = info.num_cores, info.num_subcores
    nw = nc * ns
    per_w = rows // nw
    n_chunks = per_w // _RC
    mesh = plsc.VectorSubcoreMesh(core_axis_name="c", subcore_axis_name="s")

    @functools.partial(
        pl.kernel,
        mesh=mesh,
        out_type=jax.ShapeDtypeStruct((rows, _D), jnp.float32),
        scratch_types=[
            pltpu.VMEM((_RC,), jnp.int32),
            pltpu.VMEM((_RC, _D), jnp.float32),
            pltpu.VMEM((_RC, _QW), jnp.float32),
            pltpu.SemaphoreType.DMA,
        ],
    )
    def k(table_hbm, gidx_hbm, qrep_hbm, out_hbm, idx_v, rows_v, q_v, sem):
        wid = lax.axis_index("s") * nc + lax.axis_index("c")
        base = wid * per_w

        def chunk(t, _):
            gbase = base + t * _RC
            pltpu.sync_copy(gidx_hbm.at[pl.ds(gbase, _RC)], idx_v)
            pltpu.async_copy(table_hbm.at[idx_v], rows_v, sem).wait()
            pltpu.sync_copy(qrep_hbm.at[pl.ds(gbase, _RC)], q_v)

            def sub(r, _):
                rows_v[r, 0:_QW] = rows_v[r, 0:_QW] - q_v[r, :]
                return ()

            lax.fori_loop(0, _RC, sub, ())
            pltpu.sync_copy(rows_v, out_hbm.at[pl.ds(gbase, _RC)])
            return ()

        lax.fori_loop(0, n_chunks, chunk, ())

    return k(table, gidx, qrep)


def kernel(coords, features, t_embed, queries):
    B, N, _ = coords.shape
    M = queries.shape[1]
    C = features.shape[1]
    Ct = t_embed.shape[1]

    queries_t = jnp.transpose(queries, (0, 2, 1))
    pad = _D - 3 - C - Ct
    table = jnp.concatenate(
        [coords,
         jnp.transpose(features, (0, 2, 1)),
         jnp.transpose(t_embed, (0, 2, 1)),
         jnp.zeros((B, N, pad), jnp.float32)],
        axis=-1)
    qpad = jnp.concatenate(
        [queries, jnp.zeros((B, M, _QW - 3), jnp.float32)], axis=-1)
    qrep = jnp.broadcast_to(qpad[:, :, None, :], (B, M, _K, _QW))

    gs = []
    for b in range(B):
        gidx_b = jnp.transpose(
            _ball_query(queries_t[b:b + 1], coords[b:b + 1]),
            (0, 2, 1)).reshape(M * _K)
        g_b = _sc_gather(table[b], gidx_b, qrep[b].reshape(M * _K, _QW))
        gs.append(g_b.reshape(1, M, _K, _D))
    g = jnp.concatenate(gs, axis=0)

    grouped_features = jnp.transpose(g[..., 0:3 + C], (0, 3, 1, 2))
    gt = jnp.transpose(g[..., 3 + C:3 + C + Ct], (0, 3, 1, 2))
    return (grouped_features, gt)


# speedup vs baseline: 2.9154x; 1.1023x over previous
import functools

import jax
import jax.numpy as jnp
from jax import lax
from jax.experimental import pallas as pl
from jax.experimental.pallas import tpu as pltpu
from jax.experimental.pallas import tpu_sc as plsc

_RADIUS2 = 0.1 * 0.1
_K = 32


_MT = 128
_NPC = 2048
_NS = 256
_T = 6


def _ballq_kernel(q_ref, c_ref, lt_ref, out_ref, acc_s, cm_s,
                  *, n_points, k, mt, npc):
    b = pl.program_id(0)
    i = pl.program_id(2)
    n_chunks = n_points // npc

    @pl.when(i == 0)
    def _():
        acc_s[...] = jnp.zeros((k, mt), jnp.float32)
        cm_s[...] = jnp.zeros((1, mt), jnp.float32)

    qx = q_ref[0, 0:1, :]
    qy = q_ref[0, 1:2, :]
    qz = q_ref[0, 2:3, :]
    ik = lax.broadcasted_iota(jnp.int32, (k, mt), 0).astype(jnp.float32)
    nsf = jnp.float32(_NS)

    def sub_lc(s):
        lo, hi = s * _NS, (s + 1) * _NS
        dx = c_ref[0, lo:hi, 0:1] - qx
        dy = c_ref[0, lo:hi, 1:2] - qy
        dz = c_ref[0, lo:hi, 2:3] - qz
        d2 = dx * dx + dy * dy + dz * dz
        msk = jnp.where(d2 <= _RADIUS2, 1.0, 0.0).astype(jnp.float32)
        return jnp.dot(lt_ref[...], msk, preferred_element_type=jnp.float32)

    cm0 = cm_s[...]
    nmax = jnp.zeros((1, mt), jnp.float32)
    cmv = cm0
    for s in range(npc // _NS):
        lc = sub_lc(s)
        nnz = lc[_NS - 1:_NS, :]
        con = jnp.where(ik >= cmv + nnz, nsf, 0.0)
        for r in range(_T):
            p = jnp.sum(jnp.where(lc <= jnp.float32(r), 1.0, 0.0),
                        axis=0, keepdims=True)
            con = jnp.where(ik == cmv + jnp.float32(r), p, con)
        acc_s[...] += con
        nmax = jnp.maximum(nmax, nnz)
        cmv = cmv + nnz
    cm_s[...] = cmv

    tmax = jnp.max(nmax)

    def more(w):
        return w.astype(jnp.float32) * jnp.float32(_T) < tmax

    def wave(w):
        basef = w.astype(jnp.float32) * jnp.float32(_T)
        cmw = cm0
        for s in range(npc // _NS):
            lc = sub_lc(s)
            nnz = lc[_NS - 1:_NS, :]
            delta = jnp.zeros((k, mt), jnp.float32)
            for dr in range(_T):
                r = basef + jnp.float32(dr)
                p = jnp.sum(jnp.where(lc <= r, 1.0, 0.0),
                            axis=0, keepdims=True)
                delta = jnp.where((ik == cmw + r) & (nnz > r), p, delta)
            acc_s[...] += delta
            cmw = cmw + nnz
        return w + 1

    lax.while_loop(more, wave, jnp.int32(1))

    @pl.when(i == n_chunks - 1)
    def _():
        acc = acc_s[...]
        nf = jnp.float32(n_points)
        first = acc[0:1, :]
        first = jnp.where(first >= nf, 0.0, first)
        idx = jnp.where(acc >= nf, first, acc)
        out_ref[0, :, :] = idx.astype(jnp.int32) + b * n_points


def _ball_query(queries_t, coords):
    B, N, _ = coords.shape
    M = queries_t.shape[2]
    ra = lax.broadcasted_iota(jnp.int32, (_NS, _NS), 0)
    rb = lax.broadcasted_iota(jnp.int32, (_NS, _NS), 1)
    lt = jnp.where(rb <= ra, 1.0, 0.0).astype(jnp.float32)
    grid = (B, M // _MT, N // _NPC)
    return pl.pallas_call(
        functools.partial(_ballq_kernel, n_points=N, k=_K, mt=_MT, npc=_NPC),
        grid=grid,
        in_specs=[
            pl.BlockSpec((1, 3, _MT), lambda b, m, i: (b, 0, m)),
            pl.BlockSpec((1, _NPC, 3), lambda b, m, i: (b, i, 0)),
            pl.BlockSpec((_NS, _NS), lambda b, m, i: (0, 0)),
        ],
        out_specs=pl.BlockSpec((1, _K, _MT), lambda b, m, i: (b, 0, m)),
        out_shape=jax.ShapeDtypeStruct((B, _K, M), jnp.int32),
        scratch_shapes=[
            pltpu.VMEM((_K, _MT), jnp.float32),
            pltpu.VMEM((1, _MT), jnp.float32),
        ],
    )(queries_t, coords, lt)


_D = 128
_QW = 16
_RC = 128


def _sc_gather(table, gidx, qrep):
    rows = gidx.shape[0]
    info = plsc.get_sparse_core_info()
    nc, ns = info.num_cores, info.num_subcores
    nw = nc * ns
    per_w = rows // nw
    n_chunks = per_w // _RC
    mesh = plsc.VectorSubcoreMesh(core_axis_name="c", subcore_axis_name="s")

    @functools.partial(
        pl.kernel,
        mesh=mesh,
        out_type=jax.ShapeDtypeStruct((rows, _D), jnp.float32),
        scratch_types=[
            pltpu.VMEM((_RC,), jnp.int32),
            pltpu.VMEM((_RC, _D), jnp.float32),
            pltpu.VMEM((_RC, _QW), jnp.float32),
            pltpu.SemaphoreType.DMA,
        ],
    )
    def k(table_hbm, gidx_hbm, qrep_hbm, out_hbm, idx_v, rows_v, q_v, sem):
        wid = lax.axis_index("s") * nc + lax.axis_index("c")
        base = wid * per_w

        def chunk(t, _):
            gbase = base + t * _RC
            pltpu.sync_copy(gidx_hbm.at[pl.ds(gbase, _RC)], idx_v)
            pltpu.async_copy(table_hbm.at[idx_v], rows_v, sem).wait()
            pltpu.sync_copy(qrep_hbm.at[pl.ds(gbase, _RC)], q_v)

            def sub(r, _):
                rows_v[r, 0:_QW] = rows_v[r, 0:_QW] - q_v[r, :]
                return ()

            lax.fori_loop(0, _RC, sub, ())
            pltpu.sync_copy(rows_v, out_hbm.at[pl.ds(gbase, _RC)])
            return ()

        lax.fori_loop(0, n_chunks, chunk, ())

    return k(table, gidx, qrep)


def kernel(coords, features, t_embed, queries):
    B, N, _ = coords.shape
    M = queries.shape[1]
    C = features.shape[1]
    Ct = t_embed.shape[1]

    queries_t = jnp.transpose(queries, (0, 2, 1))
    pad = _D - 3 - C - Ct
    table = jnp.concatenate(
        [coords,
         jnp.transpose(features, (0, 2, 1)),
         jnp.transpose(t_embed, (0, 2, 1)),
         jnp.zeros((B, N, pad), jnp.float32)],
        axis=-1)
    qpad = jnp.concatenate(
        [queries, jnp.zeros((B, M, _QW - 3), jnp.float32)], axis=-1)
    qrep = jnp.broadcast_to(qpad[:, :, None, :], (B, M, _K, _QW))

    gs = []
    for b in range(B):
        gidx_b = jnp.transpose(
            _ball_query(queries_t[b:b + 1], coords[b:b + 1]),
            (0, 2, 1)).reshape(M * _K)
        g_b = _sc_gather(table[b], gidx_b, qrep[b].reshape(M * _K, _QW))
        gs.append(g_b.reshape(1, M, _K, _D))
    g = jnp.concatenate(gs, axis=0)

    grouped_features = jnp.transpose(g[..., 0:3 + C], (0, 3, 1, 2))
    gt = jnp.transpose(g[..., 3 + C:3 + C + Ct], (0, 3, 1, 2))
    return (grouped_features, gt)


# speedup vs baseline: 3.0520x; 1.0469x over previous
import functools

import jax
import jax.numpy as jnp
from jax import lax
from jax.experimental import pallas as pl
from jax.experimental.pallas import tpu as pltpu
from jax.experimental.pallas import tpu_sc as plsc

_RADIUS2 = 0.1 * 0.1
_K = 32


_MT = 128
_NPC = 4096
_NS = 256
_T = 6


def _ballq_kernel(q_ref, c_ref, lt_ref, out_ref, acc_s, cm_s,
                  *, n_points, k, mt, npc):
    b = pl.program_id(0)
    i = pl.program_id(2)
    n_chunks = n_points // npc

    @pl.when(i == 0)
    def _():
        acc_s[...] = jnp.zeros((k, mt), jnp.float32)
        cm_s[...] = jnp.zeros((1, mt), jnp.float32)

    qx = q_ref[0, 0:1, :]
    qy = q_ref[0, 1:2, :]
    qz = q_ref[0, 2:3, :]
    ik = lax.broadcasted_iota(jnp.int32, (k, mt), 0).astype(jnp.float32)
    nsf = jnp.float32(_NS)

    def sub_lc(s):
        lo, hi = s * _NS, (s + 1) * _NS
        dx = c_ref[0, lo:hi, 0:1] - qx
        dy = c_ref[0, lo:hi, 1:2] - qy
        dz = c_ref[0, lo:hi, 2:3] - qz
        d2 = dx * dx + dy * dy + dz * dz
        msk = jnp.where(d2 <= _RADIUS2, 1.0, 0.0).astype(jnp.float32)
        return jnp.dot(lt_ref[...], msk, preferred_element_type=jnp.float32)

    cm0 = cm_s[...]
    nmax = jnp.zeros((1, mt), jnp.float32)
    cmv = cm0
    for s in range(npc // _NS):
        lc = sub_lc(s)
        nnz = lc[_NS - 1:_NS, :]
        con = jnp.where(ik >= cmv + nnz, nsf, 0.0)
        for r in range(_T):
            p = jnp.sum(jnp.where(lc <= jnp.float32(r), 1.0, 0.0),
                        axis=0, keepdims=True)
            con = jnp.where(ik == cmv + jnp.float32(r), p, con)
        acc_s[...] += con
        nmax = jnp.maximum(nmax, nnz)
        cmv = cmv + nnz
    cm_s[...] = cmv

    tmax = jnp.max(nmax)

    def more(w):
        return w.astype(jnp.float32) * jnp.float32(_T) < tmax

    def wave(w):
        basef = w.astype(jnp.float32) * jnp.float32(_T)
        cmw = cm0
        for s in range(npc // _NS):
            lc = sub_lc(s)
            nnz = lc[_NS - 1:_NS, :]
            delta = jnp.zeros((k, mt), jnp.float32)
            for dr in range(_T):
                r = basef + jnp.float32(dr)
                p = jnp.sum(jnp.where(lc <= r, 1.0, 0.0),
                            axis=0, keepdims=True)
                delta = jnp.where((ik == cmw + r) & (nnz > r), p, delta)
            acc_s[...] += delta
            cmw = cmw + nnz
        return w + 1

    lax.while_loop(more, wave, jnp.int32(1))

    @pl.when(i == n_chunks - 1)
    def _():
        acc = acc_s[...]
        nf = jnp.float32(n_points)
        first = acc[0:1, :]
        first = jnp.where(first >= nf, 0.0, first)
        idx = jnp.where(acc >= nf, first, acc)
        out_ref[0, :, :] = idx.astype(jnp.int32) + b * n_points


def _ball_query(queries_t, coords):
    B, N, _ = coords.shape
    M = queries_t.shape[2]
    ra = lax.broadcasted_iota(jnp.int32, (_NS, _NS), 0)
    rb = lax.broadcasted_iota(jnp.int32, (_NS, _NS), 1)
    lt = jnp.where(rb <= ra, 1.0, 0.0).astype(jnp.float32)
    grid = (B, M // _MT, N // _NPC)
    return pl.pallas_call(
        functools.partial(_ballq_kernel, n_points=N, k=_K, mt=_MT, npc=_NPC),
        grid=grid,
        in_specs=[
            pl.BlockSpec((1, 3, _MT), lambda b, m, i: (b, 0, m)),
            pl.BlockSpec((1, _NPC, 3), lambda b, m, i: (b, i, 0)),
            pl.BlockSpec((_NS, _NS), lambda b, m, i: (0, 0)),
        ],
        out_specs=pl.BlockSpec((1, _K, _MT), lambda b, m, i: (b, 0, m)),
        out_shape=jax.ShapeDtypeStruct((B, _K, M), jnp.int32),
        scratch_shapes=[
            pltpu.VMEM((_K, _MT), jnp.float32),
            pltpu.VMEM((1, _MT), jnp.float32),
        ],
    )(queries_t, coords, lt)


_D = 128
_QW = 16
_RC = 128


def _sc_gather(table, gidx, qrep):
    rows = gidx.shape[0]
    info = plsc.get_sparse_core_info()
    nc, ns = info.num_cores, info.num_subcores
    nw = nc * ns
    per_w = rows // nw
    n_chunks = per_w // _RC
    mesh = plsc.VectorSubcoreMesh(core_axis_name="c", subcore_axis_name="s")

    @functools.partial(
        pl.kernel,
        mesh=mesh,
        out_type=jax.ShapeDtypeStruct((rows, _D), jnp.float32),
        scratch_types=[
            pltpu.VMEM((_RC,), jnp.int32),
            pltpu.VMEM((_RC, _D), jnp.float32),
            pltpu.VMEM((_RC, _QW), jnp.float32),
            pltpu.SemaphoreType.DMA,
        ],
    )
    def k(table_hbm, gidx_hbm, qrep_hbm, out_hbm, idx_v, rows_v, q_v, sem):
        wid = lax.axis_index("s") * nc + lax.axis_index("c")
        base = wid * per_w

        def chunk(t, _):
            gbase = base + t * _RC
            pltpu.sync_copy(gidx_hbm.at[pl.ds(gbase, _RC)], idx_v)
            pltpu.async_copy(table_hbm.at[idx_v], rows_v, sem).wait()
            pltpu.sync_copy(qrep_hbm.at[pl.ds(gbase, _RC)], q_v)

            def sub(r, _):
                rows_v[r, 0:_QW] = rows_v[r, 0:_QW] - q_v[r, :]
                return ()

            lax.fori_loop(0, _RC, sub, ())
            pltpu.sync_copy(rows_v, out_hbm.at[pl.ds(gbase, _RC)])
            return ()

        lax.fori_loop(0, n_chunks, chunk, ())

    return k(table, gidx, qrep)


def kernel(coords, features, t_embed, queries):
    B, N, _ = coords.shape
    M = queries.shape[1]
    C = features.shape[1]
    Ct = t_embed.shape[1]

    queries_t = jnp.transpose(queries, (0, 2, 1))
    pad = _D - 3 - C - Ct
    table = jnp.concatenate(
        [coords,
         jnp.transpose(features, (0, 2, 1)),
         jnp.transpose(t_embed, (0, 2, 1)),
         jnp.zeros((B, N, pad), jnp.float32)],
        axis=-1)
    qpad = jnp.concatenate(
        [queries, jnp.zeros((B, M, _QW - 3), jnp.float32)], axis=-1)
    qrep = jnp.broadcast_to(qpad[:, :, None, :], (B, M, _K, _QW))

    gs = []
    for b in range(B):
        gidx_b = jnp.transpose(
            _ball_query(queries_t[b:b + 1], coords[b:b + 1]),
            (0, 2, 1)).reshape(M * _K)
        g_b = _sc_gather(table[b], gidx_b, qrep[b].reshape(M * _K, _QW))
        gs.append(g_b.reshape(1, M, _K, _D))
    g = jnp.concatenate(gs, axis=0)

    grouped_features = jnp.transpose(g[..., 0:3 + C], (0, 3, 1, 2))
    gt = jnp.transpose(g[..., 3 + C:3 + C + Ct], (0, 3, 1, 2))
    return (grouped_features, gt)


# speedup vs baseline: 3.1036x; 1.0169x over previous
import functools

import jax
import jax.numpy as jnp
from jax import lax
from jax.experimental import pallas as pl
from jax.experimental.pallas import tpu as pltpu
from jax.experimental.pallas import tpu_sc as plsc

_RADIUS2 = 0.1 * 0.1
_K = 32


_MT = 128
_NPC = 4096
_NS = 256
_T = 7


def _ballq_kernel(q_ref, c_ref, lt_ref, out_ref, acc_s, cm_s,
                  *, n_points, k, mt, npc):
    b = pl.program_id(0)
    i = pl.program_id(2)
    n_chunks = n_points // npc

    @pl.when(i == 0)
    def _():
        acc_s[...] = jnp.zeros((k, mt), jnp.float32)
        cm_s[...] = jnp.zeros((1, mt), jnp.float32)

    qx = q_ref[0, 0:1, :]
    qy = q_ref[0, 1:2, :]
    qz = q_ref[0, 2:3, :]
    ik = lax.broadcasted_iota(jnp.int32, (k, mt), 0).astype(jnp.float32)
    nsf = jnp.float32(_NS)

    def sub_lc(s):
        lo, hi = s * _NS, (s + 1) * _NS
        dx = c_ref[0, lo:hi, 0:1] - qx
        dy = c_ref[0, lo:hi, 1:2] - qy
        dz = c_ref[0, lo:hi, 2:3] - qz
        d2 = dx * dx + dy * dy + dz * dz
        msk = jnp.where(d2 <= _RADIUS2, 1.0, 0.0).astype(jnp.float32)
        return jnp.dot(lt_ref[...], msk, preferred_element_type=jnp.float32)

    cm0 = cm_s[...]
    nmax = jnp.zeros((1, mt), jnp.float32)
    cmv = cm0
    for s in range(npc // _NS):
        lc = sub_lc(s)
        nnz = lc[_NS - 1:_NS, :]
        con = jnp.where(ik >= cmv + nnz, nsf, 0.0)
        for r in range(_T):
            p = jnp.sum(jnp.where(lc <= jnp.float32(r), 1.0, 0.0),
                        axis=0, keepdims=True)
            con = jnp.where(ik == cmv + jnp.float32(r), p, con)
        acc_s[...] += con
        nmax = jnp.maximum(nmax, nnz)
        cmv = cmv + nnz
    cm_s[...] = cmv

    tmax = jnp.max(nmax)

    def more(w):
        return w.astype(jnp.float32) * jnp.float32(_T) < tmax

    def wave(w):
        basef = w.astype(jnp.float32) * jnp.float32(_T)
        cmw = cm0
        for s in range(npc // _NS):
            lc = sub_lc(s)
            nnz = lc[_NS - 1:_NS, :]
            delta = jnp.zeros((k, mt), jnp.float32)
            for dr in range(_T):
                r = basef + jnp.float32(dr)
                p = jnp.sum(jnp.where(lc <= r, 1.0, 0.0),
                            axis=0, keepdims=True)
                delta = jnp.where((ik == cmw + r) & (nnz > r), p, delta)
            acc_s[...] += delta
            cmw = cmw + nnz
        return w + 1

    lax.while_loop(more, wave, jnp.int32(1))

    @pl.when(i == n_chunks - 1)
    def _():
        acc = acc_s[...]
        nf = jnp.float32(n_points)
        first = acc[0:1, :]
        first = jnp.where(first >= nf, 0.0, first)
        idx = jnp.where(acc >= nf, first, acc)
        out_ref[0, :, :] = idx.astype(jnp.int32) + b * n_points


def _ball_query(queries_t, coords):
    B, N, _ = coords.shape
    M = queries_t.shape[2]
    ra = lax.broadcasted_iota(jnp.int32, (_NS, _NS), 0)
    rb = lax.broadcasted_iota(jnp.int32, (_NS, _NS), 1)
    lt = jnp.where(rb <= ra, 1.0, 0.0).astype(jnp.float32)
    grid = (B, M // _MT, N // _NPC)
    return pl.pallas_call(
        functools.partial(_ballq_kernel, n_points=N, k=_K, mt=_MT, npc=_NPC),
        grid=grid,
        in_specs=[
            pl.BlockSpec((1, 3, _MT), lambda b, m, i: (b, 0, m)),
            pl.BlockSpec((1, _NPC, 3), lambda b, m, i: (b, i, 0)),
            pl.BlockSpec((_NS, _NS), lambda b, m, i: (0, 0)),
        ],
        out_specs=pl.BlockSpec((1, _K, _MT), lambda b, m, i: (b, 0, m)),
        out_shape=jax.ShapeDtypeStruct((B, _K, M), jnp.int32),
        scratch_shapes=[
            pltpu.VMEM((_K, _MT), jnp.float32),
            pltpu.VMEM((1, _MT), jnp.float32),
        ],
    )(queries_t, coords, lt)


_D = 128
_QW = 16
_RC = 128


def _sc_gather(table, gidx, qrep):
    rows = gidx.shape[0]
    info = plsc.get_sparse_core_info()
    nc, ns = info.num_cores, info.num_subcores
    nw = nc * ns
    per_w = rows // nw
    n_chunks = per_w // _RC
    mesh = plsc.VectorSubcoreMesh(core_axis_name="c", subcore_axis_name="s")

    @functools.partial(
        pl.kernel,
        mesh=mesh,
        out_type=jax.ShapeDtypeStruct((rows, _D), jnp.float32),
        scratch_types=[
            pltpu.VMEM((_RC,), jnp.int32),
            pltpu.VMEM((_RC, _D), jnp.float32),
            pltpu.VMEM((_RC, _QW), jnp.float32),
            pltpu.SemaphoreType.DMA,
        ],
    )
    def k(table_hbm, gidx_hbm, qrep_hbm, out_hbm, idx_v, rows_v, q_v, sem):
        wid = lax.axis_index("s") * nc + lax.axis_index("c")
        base = wid * per_w

        def chunk(t, _):
            gbase = base + t * _RC
            pltpu.sync_copy(gidx_hbm.at[pl.ds(gbase, _RC)], idx_v)
            pltpu.async_copy(table_hbm.at[idx_v], rows_v, sem).wait()
            pltpu.sync_copy(qrep_hbm.at[pl.ds(gbase, _RC)], q_v)

            def sub(r, _):
                rows_v[r, 0:_QW] = rows_v[r, 0:_QW] - q_v[r, :]
                return ()

            lax.fori_loop(0, _RC, sub, ())
            pltpu.sync_copy(rows_v, out_hbm.at[pl.ds(gbase, _RC)])
            return ()

        lax.fori_loop(0, n_chunks, chunk, ())

    return k(table, gidx, qrep)


def kernel(coords, features, t_embed, queries):
    B, N, _ = coords.shape
    M = queries.shape[1]
    C = features.shape[1]
    Ct = t_embed.shape[1]

    queries_t = jnp.transpose(queries, (0, 2, 1))
    pad = _D - 3 - C - Ct
    table = jnp.concatenate(
        [coords,
         jnp.transpose(features, (0, 2, 1)),
         jnp.transpose(t_embed, (0, 2, 1)),
         jnp.zeros((B, N, pad), jnp.float32)],
        axis=-1)
    qpad = jnp.concatenate(
        [queries, jnp.zeros((B, M, _QW - 3), jnp.float32)], axis=-1)
    qrep = jnp.broadcast_to(qpad[:, :, None, :], (B, M, _K, _QW))

    gs = []
    for b in range(B):
        gidx_b = jnp.transpose(
            _ball_query(queries_t[b:b + 1], coords[b:b + 1]),
            (0, 2, 1)).reshape(M * _K)
        g_b = _sc_gather(table[b], gidx_b, qrep[b].reshape(M * _K, _QW))
        gs.append(g_b.reshape(1, M, _K, _D))
    g = jnp.concatenate(gs, axis=0)

    grouped_features = jnp.transpose(g[..., 0:3 + C], (0, 3, 1, 2))
    gt = jnp.transpose(g[..., 3 + C:3 + C + Ct], (0, 3, 1, 2))
    return (grouped_features, gt)


# speedup vs baseline: 3.1786x; 1.0241x over previous
import functools

import jax
import jax.numpy as jnp
from jax import lax
from jax.experimental import pallas as pl
from jax.experimental.pallas import tpu as pltpu
from jax.experimental.pallas import tpu_sc as plsc

_RADIUS2 = 0.1 * 0.1
_K = 32


_MT = 128
_NPC = 8192
_NS = 256
_T = 7


def _ballq_kernel(q_ref, c_ref, lt_ref, out_ref, acc_s, cm_s,
                  *, n_points, k, mt, npc):
    b = pl.program_id(0)
    i = pl.program_id(2)
    n_chunks = n_points // npc

    @pl.when(i == 0)
    def _():
        acc_s[...] = jnp.zeros((k, mt), jnp.float32)
        cm_s[...] = jnp.zeros((1, mt), jnp.float32)

    qx = q_ref[0, 0:1, :]
    qy = q_ref[0, 1:2, :]
    qz = q_ref[0, 2:3, :]
    ik = lax.broadcasted_iota(jnp.int32, (k, mt), 0).astype(jnp.float32)
    nsf = jnp.float32(_NS)

    def sub_lc(s):
        lo, hi = s * _NS, (s + 1) * _NS
        dx = c_ref[0, lo:hi, 0:1] - qx
        dy = c_ref[0, lo:hi, 1:2] - qy
        dz = c_ref[0, lo:hi, 2:3] - qz
        d2 = dx * dx + dy * dy + dz * dz
        msk = jnp.where(d2 <= _RADIUS2, 1.0, 0.0).astype(jnp.float32)
        return jnp.dot(lt_ref[...], msk, preferred_element_type=jnp.float32)

    cm0 = cm_s[...]
    nmax = jnp.zeros((1, mt), jnp.float32)
    cmv = cm0
    for s in range(npc // _NS):
        lc = sub_lc(s)
        nnz = lc[_NS - 1:_NS, :]
        con = jnp.where(ik >= cmv + nnz, nsf, 0.0)
        for r in range(_T):
            p = jnp.sum(jnp.where(lc <= jnp.float32(r), 1.0, 0.0),
                        axis=0, keepdims=True)
            con = jnp.where(ik == cmv + jnp.float32(r), p, con)
        acc_s[...] += con
        nmax = jnp.maximum(nmax, nnz)
        cmv = cmv + nnz
    cm_s[...] = cmv

    tmax = jnp.max(nmax)

    def more(w):
        return w.astype(jnp.float32) * jnp.float32(_T) < tmax

    def wave(w):
        basef = w.astype(jnp.float32) * jnp.float32(_T)
        cmw = cm0
        for s in range(npc // _NS):
            lc = sub_lc(s)
            nnz = lc[_NS - 1:_NS, :]
            delta = jnp.zeros((k, mt), jnp.float32)
            for dr in range(_T):
                r = basef + jnp.float32(dr)
                p = jnp.sum(jnp.where(lc <= r, 1.0, 0.0),
                            axis=0, keepdims=True)
                delta = jnp.where((ik == cmw + r) & (nnz > r), p, delta)
            acc_s[...] += delta
            cmw = cmw + nnz
        return w + 1

    lax.while_loop(more, wave, jnp.int32(1))

    @pl.when(i == n_chunks - 1)
    def _():
        acc = acc_s[...]
        nf = jnp.float32(n_points)
        first = acc[0:1, :]
        first = jnp.where(first >= nf, 0.0, first)
        idx = jnp.where(acc >= nf, first, acc)
        out_ref[0, :, :] = idx.astype(jnp.int32) + b * n_points


def _ball_query(queries_t, coords):
    B, N, _ = coords.shape
    M = queries_t.shape[2]
    ra = lax.broadcasted_iota(jnp.int32, (_NS, _NS), 0)
    rb = lax.broadcasted_iota(jnp.int32, (_NS, _NS), 1)
    lt = jnp.where(rb <= ra, 1.0, 0.0).astype(jnp.float32)
    grid = (B, M // _MT, N // _NPC)
    return pl.pallas_call(
        functools.partial(_ballq_kernel, n_points=N, k=_K, mt=_MT, npc=_NPC),
        grid=grid,
        in_specs=[
            pl.BlockSpec((1, 3, _MT), lambda b, m, i: (b, 0, m)),
            pl.BlockSpec((1, _NPC, 3), lambda b, m, i: (b, i, 0)),
            pl.BlockSpec((_NS, _NS), lambda b, m, i: (0, 0)),
        ],
        out_specs=pl.BlockSpec((1, _K, _MT), lambda b, m, i: (b, 0, m)),
        out_shape=jax.ShapeDtypeStruct((B, _K, M), jnp.int32),
        scratch_shapes=[
            pltpu.VMEM((_K, _MT), jnp.float32),
            pltpu.VMEM((1, _MT), jnp.float32),
        ],
    )(queries_t, coords, lt)


_D = 128
_QW = 16
_RC = 128


def _sc_gather(table, gidx, qrep):
    rows = gidx.shape[0]
    info = plsc.get_sparse_core_info()
    nc, ns = info.num_cores, info.num_subcores
    nw = nc * ns
    per_w = rows // nw
    n_chunks = per_w // _RC
    mesh = plsc.VectorSubcoreMesh(core_axis_name="c", subcore_axis_name="s")

    @functools.partial(
        pl.kernel,
        mesh=mesh,
        out_type=jax.ShapeDtypeStruct((rows, _D), jnp.float32),
        scratch_types=[
            pltpu.VMEM((_RC,), jnp.int32),
            pltpu.VMEM((_RC, _D), jnp.float32),
            pltpu.VMEM((_RC, _QW), jnp.float32),
            pltpu.SemaphoreType.DMA,
        ],
    )
    def k(table_hbm, gidx_hbm, qrep_hbm, out_hbm, idx_v, rows_v, q_v, sem):
        wid = lax.axis_index("s") * nc + lax.axis_index("c")
        base = wid * per_w

        def chunk(t, _):
            gbase = base + t * _RC
            pltpu.sync_copy(gidx_hbm.at[pl.ds(gbase, _RC)], idx_v)
            pltpu.async_copy(table_hbm.at[idx_v], rows_v, sem).wait()
            pltpu.sync_copy(qrep_hbm.at[pl.ds(gbase, _RC)], q_v)

            def sub(r, _):
                rows_v[r, 0:_QW] = rows_v[r, 0:_QW] - q_v[r, :]
                return ()

            lax.fori_loop(0, _RC, sub, ())
            pltpu.sync_copy(rows_v, out_hbm.at[pl.ds(gbase, _RC)])
            return ()

        lax.fori_loop(0, n_chunks, chunk, ())

    return k(table, gidx, qrep)


def kernel(coords, features, t_embed, queries):
    B, N, _ = coords.shape
    M = queries.shape[1]
    C = features.shape[1]
    Ct = t_embed.shape[1]

    queries_t = jnp.transpose(queries, (0, 2, 1))
    pad = _D - 3 - C - Ct
    table = jnp.concatenate(
        [coords,
         jnp.transpose(features, (0, 2, 1)),
         jnp.transpose(t_embed, (0, 2, 1)),
         jnp.zeros((B, N, pad), jnp.float32)],
        axis=-1)
    qpad = jnp.concatenate(
        [queries, jnp.zeros((B, M, _QW - 3), jnp.float32)], axis=-1)
    qrep = jnp.broadcast_to(qpad[:, :, None, :], (B, M, _K, _QW))

    gs = []
    for b in range(B):
        gidx_b = jnp.transpose(
            _ball_query(queries_t[b:b + 1], coords[b:b + 1]),
            (0, 2, 1)).reshape(M * _K)
        g_b = _sc_gather(table[b], gidx_b, qrep[b].reshape(M * _K, _QW))
        gs.append(g_b.reshape(1, M, _K, _D))
    g = jnp.concatenate(gs, axis=0)

    grouped_features = jnp.transpose(g[..., 0:3 + C], (0, 3, 1, 2))
    gt = jnp.transpose(g[..., 3 + C:3 + C + Ct], (0, 3, 1, 2))
    return (grouped_features, gt)


# speedup vs baseline: 3.2720x; 1.0294x over previous
import functools

import jax
import jax.numpy as jnp
from jax import lax
from jax.experimental import pallas as pl
from jax.experimental.pallas import tpu as pltpu
from jax.experimental.pallas import tpu_sc as plsc

_RADIUS2 = 0.1 * 0.1
_K = 32


_MT = 128
_NPC = 8192
_NS = 256
_T = 7


def _ballq_kernel(q_ref, c_ref, lt_ref, out_ref, acc_s, cm_s,
                  *, n_points, k, mt, npc):
    b = pl.program_id(0)
    i = pl.program_id(2)
    n_chunks = n_points // npc

    @pl.when(i == 0)
    def _():
        acc_s[...] = jnp.zeros((k, mt), jnp.float32)
        cm_s[...] = jnp.zeros((1, mt), jnp.float32)

    qx = q_ref[0, 0:1, :]
    qy = q_ref[0, 1:2, :]
    qz = q_ref[0, 2:3, :]
    ik = lax.broadcasted_iota(jnp.int32, (k, mt), 0).astype(jnp.float32)
    nsf = jnp.float32(_NS)

    def sub_lc(s):
        lo, hi = s * _NS, (s + 1) * _NS
        dx = c_ref[0, lo:hi, 0:1] - qx
        dy = c_ref[0, lo:hi, 1:2] - qy
        dz = c_ref[0, lo:hi, 2:3] - qz
        d2 = dx * dx + dy * dy + dz * dz
        msk = jnp.where(d2 <= _RADIUS2, 1.0, 0.0).astype(jnp.float32)
        return jnp.dot(lt_ref[...], msk, preferred_element_type=jnp.float32)

    cm0 = cm_s[...]
    nmax = jnp.zeros((1, mt), jnp.float32)
    cmv = cm0
    for s in range(npc // _NS):
        lc = sub_lc(s)
        nnz = lc[_NS - 1:_NS, :]
        con = jnp.where(ik >= cmv + nnz, nsf, 0.0)
        for r in range(_T):
            p = jnp.sum(jnp.where(lc <= jnp.float32(r), 1.0, 0.0),
                        axis=0, keepdims=True)
            con = jnp.where(ik == cmv + jnp.float32(r), p, con)
        acc_s[...] += con
        nmax = jnp.maximum(nmax, nnz)
        cmv = cmv + nnz
    cm_s[...] = cmv

    tmax = jnp.max(nmax)

    def more(w):
        return w.astype(jnp.float32) * jnp.float32(_T) < tmax

    def wave(w):
        basef = w.astype(jnp.float32) * jnp.float32(_T)
        cmw = cm0
        for s in range(npc // _NS):
            lc = sub_lc(s)
            nnz = lc[_NS - 1:_NS, :]
            delta = jnp.zeros((k, mt), jnp.float32)
            for dr in range(_T):
                r = basef + jnp.float32(dr)
                p = jnp.sum(jnp.where(lc <= r, 1.0, 0.0),
                            axis=0, keepdims=True)
                delta = jnp.where((ik == cmw + r) & (nnz > r), p, delta)
            acc_s[...] += delta
            cmw = cmw + nnz
        return w + 1

    lax.while_loop(more, wave, jnp.int32(1))

    @pl.when(i == n_chunks - 1)
    def _():
        acc = acc_s[...]
        nf = jnp.float32(n_points)
        first = acc[0:1, :]
        first = jnp.where(first >= nf, 0.0, first)
        idx = jnp.where(acc >= nf, first, acc)
        out_ref[0, :, :] = idx.astype(jnp.int32) + b * n_points


def _ball_query(queries_t, coords):
    B, N, _ = coords.shape
    M = queries_t.shape[2]
    ra = lax.broadcasted_iota(jnp.int32, (_NS, _NS), 0)
    rb = lax.broadcasted_iota(jnp.int32, (_NS, _NS), 1)
    lt = jnp.where(rb <= ra, 1.0, 0.0).astype(jnp.float32)
    grid = (B, M // _MT, N // _NPC)
    return pl.pallas_call(
        functools.partial(_ballq_kernel, n_points=N, k=_K, mt=_MT, npc=_NPC),
        grid=grid,
        in_specs=[
            pl.BlockSpec((1, 3, _MT), lambda b, m, i: (b, 0, m)),
            pl.BlockSpec((1, _NPC, 3), lambda b, m, i: (b, i, 0)),
            pl.BlockSpec((_NS, _NS), lambda b, m, i: (0, 0)),
        ],
        out_specs=pl.BlockSpec((1, _K, _MT), lambda b, m, i: (b, 0, m)),
        out_shape=jax.ShapeDtypeStruct((B, _K, M), jnp.int32),
        scratch_shapes=[
            pltpu.VMEM((_K, _MT), jnp.float32),
            pltpu.VMEM((1, _MT), jnp.float32),
        ],
    )(queries_t, coords, lt)


_D = 128
_QW = 16
_RC = 128


def _sc_gather(table, gidx, qrep):
    rows = gidx.shape[0]
    info = plsc.get_sparse_core_info()
    nc, ns = info.num_cores, info.num_subcores
    nw = nc * ns
    per_w = rows // nw
    n_chunks = per_w // _RC
    mesh = plsc.VectorSubcoreMesh(core_axis_name="c", subcore_axis_name="s")

    @functools.partial(
        pl.kernel,
        mesh=mesh,
        out_type=jax.ShapeDtypeStruct((rows, _D), jnp.float32),
        scratch_types=[
            pltpu.VMEM((2, _RC), jnp.int32),
            pltpu.VMEM((2, _RC, _D), jnp.float32),
            pltpu.VMEM((2, _RC, _QW), jnp.float32),
            pltpu.SemaphoreType.DMA,
            pltpu.SemaphoreType.DMA,
            pltpu.SemaphoreType.DMA,
            pltpu.SemaphoreType.DMA,
        ],
    )
    def k(table_hbm, gidx_hbm, qrep_hbm, out_hbm, idx_v, rows_v, q_v,
          g0, g1, o0, o1):
        wid = lax.axis_index("s") * nc + lax.axis_index("c")
        base = wid * per_w
        gsem = (g0, g1)
        osem = (o0, o1)

        def stage(t):
            par = t % 2
            gbase = base + t * _RC
            pltpu.sync_copy(gidx_hbm.at[pl.ds(gbase, _RC)], idx_v.at[par])
            pltpu.sync_copy(qrep_hbm.at[pl.ds(gbase, _RC)], q_v.at[par])
            return pltpu.async_copy(
                table_hbm.at[idx_v.at[par]], rows_v.at[par], gsem[par])

        gh = {0: stage(0)}
        oh = {}
        for t in range(n_chunks):
            par = t % 2
            if t + 1 < n_chunks:
                if t - 1 in oh:
                    oh.pop(t - 1).wait()
                gh[t + 1] = stage(t + 1)
            gh.pop(t).wait()

            def sub(r, _):
                rows_v[par, r, 0:_QW] = (rows_v[par, r, 0:_QW]
                                         - q_v[par, r, :])
                return ()

            lax.fori_loop(0, _RC, sub, ())
            oh[t] = pltpu.async_copy(
                rows_v.at[par], out_hbm.at[pl.ds(base + t * _RC, _RC)],
                osem[par])
        for t in sorted(oh):
            oh.pop(t).wait()

    return k(table, gidx, qrep)


def kernel(coords, features, t_embed, queries):
    B, N, _ = coords.shape
    M = queries.shape[1]
    C = features.shape[1]
    Ct = t_embed.shape[1]

    queries_t = jnp.transpose(queries, (0, 2, 1))
    pad = _D - 3 - C - Ct
    table = jnp.concatenate(
        [coords,
         jnp.transpose(features, (0, 2, 1)),
         jnp.transpose(t_embed, (0, 2, 1)),
         jnp.zeros((B, N, pad), jnp.float32)],
        axis=-1)
    qpad = jnp.concatenate(
        [queries, jnp.zeros((B, M, _QW - 3), jnp.float32)], axis=-1)
    qrep = jnp.broadcast_to(qpad[:, :, None, :], (B, M, _K, _QW))

    gs = []
    for b in range(B):
        gidx_b = jnp.transpose(
            _ball_query(queries_t[b:b + 1], coords[b:b + 1]),
            (0, 2, 1)).reshape(M * _K)
        g_b = _sc_gather(table[b], gidx_b, qrep[b].reshape(M * _K, _QW))
        gs.append(g_b.reshape(1, M, _K, _D))
    g = jnp.concatenate(gs, axis=0)

    grouped_features = jnp.transpose(g[..., 0:3 + C], (0, 3, 1, 2))
    gt = jnp.transpose(g[..., 3 + C:3 + C + Ct], (0, 3, 1, 2))
    return (grouped_features, gt)


# speedup vs baseline: 3.4392x; 1.0511x over previous
import functools

import jax
import jax.numpy as jnp
from jax import lax
from jax.experimental import pallas as pl
from jax.experimental.pallas import tpu as pltpu
from jax.experimental.pallas import tpu_sc as plsc

_RADIUS2 = 0.1 * 0.1
_K = 32


_MT = 128
_NPC = 8192
_NS = 256
_T = 7


def _ballq_kernel(q_ref, c_ref, lt_ref, out_ref, acc_s, cm_s,
                  *, n_points, k, mt, npc):
    b = pl.program_id(0)
    i = pl.program_id(2)
    n_chunks = n_points // npc

    @pl.when(i == 0)
    def _():
        acc_s[...] = jnp.zeros((k, mt), jnp.float32)
        cm_s[...] = jnp.zeros((1, mt), jnp.float32)

    qx = q_ref[0, 0:1, :]
    qy = q_ref[0, 1:2, :]
    qz = q_ref[0, 2:3, :]
    ik = lax.broadcasted_iota(jnp.int32, (k, mt), 0).astype(jnp.float32)
    nsf = jnp.float32(_NS)

    def sub_lc(s):
        lo, hi = s * _NS, (s + 1) * _NS
        dx = c_ref[0, lo:hi, 0:1] - qx
        dy = c_ref[0, lo:hi, 1:2] - qy
        dz = c_ref[0, lo:hi, 2:3] - qz
        d2 = dx * dx + dy * dy + dz * dz
        msk = jnp.where(d2 <= _RADIUS2, 1.0, 0.0).astype(jnp.float32)
        return jnp.dot(lt_ref[...], msk, preferred_element_type=jnp.float32)

    cm0 = cm_s[...]
    nmax = jnp.zeros((1, mt), jnp.float32)
    cmv = cm0
    for s in range(npc // _NS):
        lc = sub_lc(s)
        nnz = lc[_NS - 1:_NS, :]
        con = jnp.where(ik >= cmv + nnz, nsf, 0.0)
        for r in range(_T):
            p = jnp.sum(jnp.where(lc <= jnp.float32(r), 1.0, 0.0),
                        axis=0, keepdims=True)
            con = jnp.where(ik == cmv + jnp.float32(r), p, con)
        acc_s[...] += con
        nmax = jnp.maximum(nmax, nnz)
        cmv = cmv + nnz
    cm_s[...] = cmv

    tmax = jnp.max(nmax)

    def more(w):
        return w.astype(jnp.float32) * jnp.float32(_T) < tmax

    def wave(w):
        basef = w.astype(jnp.float32) * jnp.float32(_T)
        cmw = cm0
        for s in range(npc // _NS):
            lc = sub_lc(s)
            nnz = lc[_NS - 1:_NS, :]
            delta = jnp.zeros((k, mt), jnp.float32)
            for dr in range(_T):
                r = basef + jnp.float32(dr)
                p = jnp.sum(jnp.where(lc <= r, 1.0, 0.0),
                            axis=0, keepdims=True)
                delta = jnp.where((ik == cmw + r) & (nnz > r), p, delta)
            acc_s[...] += delta
            cmw = cmw + nnz
        return w + 1

    lax.while_loop(more, wave, jnp.int32(1))

    @pl.when(i == n_chunks - 1)
    def _():
        acc = acc_s[...]
        nf = jnp.float32(n_points)
        first = acc[0:1, :]
        first = jnp.where(first >= nf, 0.0, first)
        idx = jnp.where(acc >= nf, first, acc)
        out_ref[0, :, :] = idx.astype(jnp.int32) + b * n_points


def _ball_query(queries_t, coords):
    B, N, _ = coords.shape
    M = queries_t.shape[2]
    ra = lax.broadcasted_iota(jnp.int32, (_NS, _NS), 0)
    rb = lax.broadcasted_iota(jnp.int32, (_NS, _NS), 1)
    lt = jnp.where(rb <= ra, 1.0, 0.0).astype(jnp.float32)
    grid = (B, M // _MT, N // _NPC)
    return pl.pallas_call(
        functools.partial(_ballq_kernel, n_points=N, k=_K, mt=_MT, npc=_NPC),
        grid=grid,
        in_specs=[
            pl.BlockSpec((1, 3, _MT), lambda b, m, i: (b, 0, m)),
            pl.BlockSpec((1, _NPC, 3), lambda b, m, i: (b, i, 0)),
            pl.BlockSpec((_NS, _NS), lambda b, m, i: (0, 0)),
        ],
        out_specs=pl.BlockSpec((1, _K, _MT), lambda b, m, i: (b, 0, m)),
        out_shape=jax.ShapeDtypeStruct((B, _K, M), jnp.int32),
        scratch_shapes=[
            pltpu.VMEM((_K, _MT), jnp.float32),
            pltpu.VMEM((1, _MT), jnp.float32),
        ],
    )(queries_t, coords, lt)


_D = 128
_QW = 16
_RC = 128


def _sc_gather(table, gidx, qpad):
    rows = gidx.shape[0]
    info = plsc.get_sparse_core_info()
    nc, ns = info.num_cores, info.num_subcores
    nw = nc * ns
    per_w = rows // nw
    n_chunks = per_w // _RC
    q_per_w = per_w // _K
    mesh = plsc.VectorSubcoreMesh(core_axis_name="c", subcore_axis_name="s")

    @functools.partial(
        pl.kernel,
        mesh=mesh,
        out_type=jax.ShapeDtypeStruct((rows, _D), jnp.float32),
        scratch_types=[
            pltpu.VMEM((2, _RC), jnp.int32),
            pltpu.VMEM((2, _RC, _D), jnp.float32),
            pltpu.VMEM((q_per_w, _QW), jnp.float32),
            pltpu.SemaphoreType.DMA,
            pltpu.SemaphoreType.DMA,
            pltpu.SemaphoreType.DMA,
            pltpu.SemaphoreType.DMA,
        ],
    )
    def k(table_hbm, gidx_hbm, qpad_hbm, out_hbm, idx_v, rows_v, q_v,
          g0, g1, o0, o1):
        wid = lax.axis_index("s") * nc + lax.axis_index("c")
        base = wid * per_w
        gsem = (g0, g1)
        osem = (o0, o1)
        pltpu.sync_copy(qpad_hbm.at[pl.ds(wid * q_per_w, q_per_w)], q_v)

        def stage(t):
            par = t % 2
            gbase = base + t * _RC
            pltpu.sync_copy(gidx_hbm.at[pl.ds(gbase, _RC)], idx_v.at[par])
            return pltpu.async_copy(
                table_hbm.at[idx_v.at[par]], rows_v.at[par], gsem[par])

        gh = {0: stage(0)}
        oh = {}
        for t in range(n_chunks):
            par = t % 2
            qoff = t * (_RC // _K)
            if t + 1 < n_chunks:
                if t - 1 in oh:
                    oh.pop(t - 1).wait()
                gh[t + 1] = stage(t + 1)
            gh.pop(t).wait()

            def sub(r, _):
                qi = qoff + lax.div(r, _K)
                rows_v[par, r, 0:_QW] = (rows_v[par, r, 0:_QW]
                                         - q_v[qi, :])
                return ()

            lax.fori_loop(0, _RC, sub, ())
            oh[t] = pltpu.async_copy(
                rows_v.at[par], out_hbm.at[pl.ds(base + t * _RC, _RC)],
                osem[par])
        for t in sorted(oh):
            oh.pop(t).wait()

    return k(table, gidx, qpad)


def kernel(coords, features, t_embed, queries):
    B, N, _ = coords.shape
    M = queries.shape[1]
    C = features.shape[1]
    Ct = t_embed.shape[1]

    queries_t = jnp.transpose(queries, (0, 2, 1))
    pad = _D - 3 - C - Ct
    table = jnp.concatenate(
        [coords,
         jnp.transpose(features, (0, 2, 1)),
         jnp.transpose(t_embed, (0, 2, 1)),
         jnp.zeros((B, N, pad), jnp.float32)],
        axis=-1)
    qpad = jnp.concatenate(
        [queries, jnp.zeros((B, M, _QW - 3), jnp.float32)], axis=-1)

    gs = []
    for b in range(B):
        gidx_b = jnp.transpose(
            _ball_query(queries_t[b:b + 1], coords[b:b + 1]),
            (0, 2, 1)).reshape(M * _K)
        g_b = _sc_gather(table[b], gidx_b, qpad[b])
        gs.append(g_b.reshape(1, M, _K, _D))
    g = jnp.concatenate(gs, axis=0)

    grouped_features = jnp.transpose(g[..., 0:3 + C], (0, 3, 1, 2))
    gt = jnp.transpose(g[..., 3 + C:3 + C + Ct], (0, 3, 1, 2))
    return (grouped_features, gt)


# speedup vs baseline: 3.4637x; 1.0071x over previous
import functools

import jax
import jax.numpy as jnp
from jax import lax
from jax.experimental import pallas as pl
from jax.experimental.pallas import tpu as pltpu
from jax.experimental.pallas import tpu_sc as plsc

_RADIUS2 = 0.1 * 0.1
_K = 32


_MT = 128
_NPC = 8192
_NS = 256
_T = 7


def _ballq_kernel(q_ref, c_ref, lt_ref, out_ref, acc_s, cm_s,
                  *, n_points, k, mt, npc):
    b = pl.program_id(0)
    i = pl.program_id(2)
    n_chunks = n_points // npc

    @pl.when(i == 0)
    def _():
        acc_s[...] = jnp.zeros((k, mt), jnp.float32)
        cm_s[...] = jnp.zeros((1, mt), jnp.float32)

    qx = q_ref[0, 0:1, :]
    qy = q_ref[0, 1:2, :]
    qz = q_ref[0, 2:3, :]
    ik = lax.broadcasted_iota(jnp.int32, (k, mt), 0).astype(jnp.float32)
    nsf = jnp.float32(_NS)

    def sub_lc(s):
        lo, hi = s * _NS, (s + 1) * _NS
        dx = c_ref[0, lo:hi, 0:1] - qx
        dy = c_ref[0, lo:hi, 1:2] - qy
        dz = c_ref[0, lo:hi, 2:3] - qz
        d2 = dx * dx + dy * dy + dz * dz
        msk = jnp.where(d2 <= _RADIUS2, 1.0, 0.0).astype(jnp.float32)
        return jnp.dot(lt_ref[...], msk, preferred_element_type=jnp.float32)

    cm0 = cm_s[...]
    nmax = jnp.zeros((1, mt), jnp.float32)
    cmv = cm0
    for s in range(npc // _NS):
        lc = sub_lc(s)
        nnz = lc[_NS - 1:_NS, :]
        con = jnp.where(ik >= cmv + nnz, nsf, 0.0)
        for r in range(_T):
            p = jnp.sum(jnp.where(lc <= jnp.float32(r), 1.0, 0.0),
                        axis=0, keepdims=True)
            con = jnp.where(ik == cmv + jnp.float32(r), p, con)
        acc_s[...] += con
        nmax = jnp.maximum(nmax, nnz)
        cmv = cmv + nnz
    cm_s[...] = cmv

    tmax = jnp.max(nmax)

    def more(w):
        return w.astype(jnp.float32) * jnp.float32(_T) < tmax

    def wave(w):
        basef = w.astype(jnp.float32) * jnp.float32(_T)
        cmw = cm0
        for s in range(npc // _NS):
            lc = sub_lc(s)
            nnz = lc[_NS - 1:_NS, :]
            delta = jnp.zeros((k, mt), jnp.float32)
            for dr in range(_T):
                r = basef + jnp.float32(dr)
                p = jnp.sum(jnp.where(lc <= r, 1.0, 0.0),
                            axis=0, keepdims=True)
                delta = jnp.where((ik == cmw + r) & (nnz > r), p, delta)
            acc_s[...] += delta
            cmw = cmw + nnz
        return w + 1

    lax.while_loop(more, wave, jnp.int32(1))

    @pl.when(i == n_chunks - 1)
    def _():
        acc = acc_s[...]
        nf = jnp.float32(n_points)
        first = acc[0:1, :]
        first = jnp.where(first >= nf, 0.0, first)
        idx = jnp.where(acc >= nf, first, acc)
        out_ref[0, :, :] = idx.astype(jnp.int32) + b * n_points


def _ball_query(queries_t, coords):
    B, N, _ = coords.shape
    M = queries_t.shape[2]
    ra = lax.broadcasted_iota(jnp.int32, (_NS, _NS), 0)
    rb = lax.broadcasted_iota(jnp.int32, (_NS, _NS), 1)
    lt = jnp.where(rb <= ra, 1.0, 0.0).astype(jnp.float32)
    grid = (B, M // _MT, N // _NPC)
    return pl.pallas_call(
        functools.partial(_ballq_kernel, n_points=N, k=_K, mt=_MT, npc=_NPC),
        grid=grid,
        in_specs=[
            pl.BlockSpec((1, 3, _MT), lambda b, m, i: (b, 0, m)),
            pl.BlockSpec((1, _NPC, 3), lambda b, m, i: (b, i, 0)),
            pl.BlockSpec((_NS, _NS), lambda b, m, i: (0, 0)),
        ],
        out_specs=pl.BlockSpec((1, _K, _MT), lambda b, m, i: (b, 0, m)),
        out_shape=jax.ShapeDtypeStruct((B, _K, M), jnp.int32),
        scratch_shapes=[
            pltpu.VMEM((_K, _MT), jnp.float32),
            pltpu.VMEM((1, _MT), jnp.float32),
        ],
    )(queries_t, coords, lt)


_D = 128
_QW = 16
_RC = 128


def _sc_gather(table, gidx, qpad):
    gidx = gidx.reshape(-1, _RC)
    rows = gidx.shape[0] * _RC
    info = plsc.get_sparse_core_info()
    nc, ns = info.num_cores, info.num_subcores
    nw = nc * ns
    per_w = rows // nw
    n_chunks = per_w // _RC
    q_per_w = per_w // _K
    mesh = plsc.VectorSubcoreMesh(core_axis_name="c", subcore_axis_name="s")

    @functools.partial(
        pl.kernel,
        mesh=mesh,
        out_type=jax.ShapeDtypeStruct((rows, _D), jnp.float32),
        scratch_types=[
            pltpu.VMEM((n_chunks, _RC), jnp.int32),
            pltpu.VMEM((2, _RC, _D), jnp.float32),
            pltpu.VMEM((q_per_w, _QW), jnp.float32),
            pltpu.SemaphoreType.DMA,
            pltpu.SemaphoreType.DMA,
            pltpu.SemaphoreType.DMA,
            pltpu.SemaphoreType.DMA,
        ],
    )
    def k(table_hbm, gidx_hbm, qpad_hbm, out_hbm, idx_v, rows_v, q_v,
          g0, g1, o0, o1):
        wid = lax.axis_index("s") * nc + lax.axis_index("c")
        base = wid * per_w
        gsem = (g0, g1)
        osem = (o0, o1)
        pltpu.sync_copy(qpad_hbm.at[pl.ds(wid * q_per_w, q_per_w)], q_v)
        pltpu.sync_copy(gidx_hbm.at[pl.ds(wid * n_chunks, n_chunks)], idx_v)

        def stage(t):
            par = t % 2
            return pltpu.async_copy(
                table_hbm.at[idx_v.at[t]], rows_v.at[par], gsem[par])

        gh = {0: stage(0)}
        oh = {}
        for t in range(n_chunks):
            par = t % 2
            qoff = t * (_RC // _K)
            if t + 1 < n_chunks:
                if t - 1 in oh:
                    oh.pop(t - 1).wait()
                gh[t + 1] = stage(t + 1)
            gh.pop(t).wait()

            def sub(r, _):
                qi = qoff + lax.div(r, _K)
                rows_v[par, r, 0:_QW] = (rows_v[par, r, 0:_QW]
                                         - q_v[qi, :])
                return ()

            lax.fori_loop(0, _RC, sub, ())
            oh[t] = pltpu.async_copy(
                rows_v.at[par], out_hbm.at[pl.ds(base + t * _RC, _RC)],
                osem[par])
        for t in sorted(oh):
            oh.pop(t).wait()

    return k(table, gidx, qpad)


def kernel(coords, features, t_embed, queries):
    B, N, _ = coords.shape
    M = queries.shape[1]
    C = features.shape[1]
    Ct = t_embed.shape[1]

    queries_t = jnp.transpose(queries, (0, 2, 1))
    pad = _D - 3 - C - Ct
    table = jnp.concatenate(
        [coords,
         jnp.transpose(features, (0, 2, 1)),
         jnp.transpose(t_embed, (0, 2, 1)),
         jnp.zeros((B, N, pad), jnp.float32)],
        axis=-1)
    qpad = jnp.concatenate(
        [queries, jnp.zeros((B, M, _QW - 3), jnp.float32)], axis=-1)

    gs = []
    for b in range(B):
        gidx_b = jnp.transpose(
            _ball_query(queries_t[b:b + 1], coords[b:b + 1]),
            (0, 2, 1)).reshape(M * _K)
        g_b = _sc_gather(table[b], gidx_b, qpad[b])
        gs.append(g_b.reshape(1, M, _K, _D))
    g = jnp.concatenate(gs, axis=0)

    grouped_features = jnp.transpose(g[..., 0:3 + C], (0, 3, 1, 2))
    gt = jnp.transpose(g[..., 3 + C:3 + C + Ct], (0, 3, 1, 2))
    return (grouped_features, gt)
